# Initial kernel scaffold; baseline (speedup 1.0000x reference)
import jax
import jax.numpy as jnp
from jax.experimental import pallas as pl


def kernel(x, edge_attr, weights, h_bias, edge_index):
    raise NotImplementedError("write your pallas kernel here")


# speedup vs baseline: 6.6492x; 6.6492x over previous
import functools

import jax
import jax.numpy as jnp
from jax import lax
from jax.experimental import pallas as pl
from jax.experimental.pallas import tpu as pltpu
from jax.experimental.pallas import tpu_sc as plsc

N = 10000
E = 320000
D = 128

NC = 2
NS = 16
CHUNK = 80
EPT = E // (NC * NS)
ITERS = EPT // CHUNK
NPAD = 10240
ROWS_PT = NPAD // NS


def _sc_a_body(x_hbm, ea_hbm, src_hbm, dst_hbm, zeros_hbm,
               outa_hbm,
               a_sh, src_idx, dst_idx, xrows, earows, sem):
    c = lax.axis_index("c")
    s = lax.axis_index("s")
    base = (c * NS + s) * EPT
    row0 = s * ROWS_PT

    pltpu.sync_copy(zeros_hbm, xrows)
    for k in range(ROWS_PT // CHUNK):
        pltpu.sync_copy(xrows, a_sh.at[pl.ds(row0 + k * CHUNK, CHUNK)])
    plsc.subcore_barrier()

    def step(j, carry):
        off = base + j * CHUNK
        pltpu.sync_copy(src_hbm.at[pl.ds(off, CHUNK)], src_idx)
        pltpu.sync_copy(dst_hbm.at[pl.ds(off, CHUNK)], dst_idx)
        pltpu.async_copy(x_hbm.at[src_idx], xrows, sem).wait()
        pltpu.sync_copy(ea_hbm.at[pl.ds(off, CHUNK)], earows)
        pltpu.sync_copy(xrows, a_sh.at[dst_idx], add=True)
        pltpu.sync_copy(earows, a_sh.at[dst_idx], add=True)
        return carry

    lax.fori_loop(0, ITERS, step, None)
    plsc.subcore_barrier()

    for k in range(ROWS_PT // CHUNK):
        r = row0 + k * CHUNK
        pltpu.sync_copy(a_sh.at[pl.ds(r, CHUNK)], xrows)
        pltpu.sync_copy(xrows, outa_hbm.at[pl.ds(c * NPAD + r, CHUNK)])


def _sc_d_body(dst_hbm, zeros_hbm, ones_hbm,
               outd_hbm,
               d_sh, dst_idx, ones_v, z16, sem):
    c = lax.axis_index("c")
    s = lax.axis_index("s")
    base = (c * NS + s) * EPT
    row0 = s * ROWS_PT

    pltpu.sync_copy(zeros_hbm, z16)
    pltpu.sync_copy(ones_hbm, ones_v)
    for k in range(ROWS_PT // CHUNK):
        pltpu.sync_copy(z16, d_sh.at[pl.ds(row0 + k * CHUNK, CHUNK)])
    plsc.subcore_barrier()

    def step(j, carry):
        off = base + j * CHUNK
        pltpu.sync_copy(dst_hbm.at[pl.ds(off, CHUNK)], dst_idx)
        pltpu.sync_copy(ones_v, d_sh.at[dst_idx], add=True)
        return carry

    lax.fori_loop(0, ITERS, step, None)
    plsc.subcore_barrier()

    for k in range(ROWS_PT // CHUNK):
        r = row0 + k * CHUNK
        pltpu.sync_copy(d_sh.at[pl.ds(r, CHUNK)], z16)
        pltpu.sync_copy(z16, outd_hbm.at[pl.ds(c * NPAD + r, CHUNK)])


@jax.jit
def _sc_aggregate(x, edge_attr, src, dst):
    zeros = jnp.zeros((CHUNK, D), jnp.float32)
    ones = jnp.ones((CHUNK, D), jnp.float32)
    mesh = plsc.VectorSubcoreMesh(core_axis_name="c", subcore_axis_name="s")
    a_fn = functools.partial(
        pl.kernel,
        _sc_a_body,
        out_type=[jax.ShapeDtypeStruct((NC * NPAD, D), jnp.float32)],
        mesh=mesh,
        scratch_types=[
            pltpu.VMEM_SHARED((NPAD, D), jnp.float32),
            pltpu.VMEM((CHUNK,), jnp.int32),
            pltpu.VMEM((CHUNK,), jnp.int32),
            pltpu.VMEM((CHUNK, D), jnp.float32),
            pltpu.VMEM((CHUNK, D), jnp.float32),
            pltpu.SemaphoreType.DMA,
        ],
    )()
    d_fn = functools.partial(
        pl.kernel,
        _sc_d_body,
        out_type=[jax.ShapeDtypeStruct((NC * NPAD, D), jnp.float32)],
        mesh=mesh,
        scratch_types=[
            pltpu.VMEM_SHARED((NPAD, D), jnp.float32),
            pltpu.VMEM((CHUNK,), jnp.int32),
            pltpu.VMEM((CHUNK, D), jnp.float32),
            pltpu.VMEM((CHUNK, D), jnp.float32),
            pltpu.SemaphoreType.DMA,
        ],
    )()
    (a_flat,) = a_fn(x, edge_attr, src, dst, zeros)
    (d_flat,) = d_fn(dst, zeros, ones)
    return a_flat, d_flat


def _tc_body(a_ref, d_ref, w_ref, b_ref, out_ref):
    a = a_ref[0] + a_ref[1]
    deg = d_ref[0, :, :1] + d_ref[1, :, :1]
    norm = jnp.where(deg > 0.0, 1.0 / jnp.sqrt(deg), 0.0)
    out_ref[...] = (
        jnp.dot(a * norm, w_ref[...], preferred_element_type=jnp.float32)
        + b_ref[...]
    )


@jax.jit
def _tc_finish(a2, d2, weights, h_bias):
    bn = 400
    return pl.pallas_call(
        _tc_body,
        grid=(N // bn,),
        in_specs=[
            pl.BlockSpec((NC, bn, D), lambda i: (0, i, 0)),
            pl.BlockSpec((NC, bn, D), lambda i: (0, i, 0)),
            pl.BlockSpec((D, D), lambda i: (0, 0)),
            pl.BlockSpec((1, D), lambda i: (0, 0)),
        ],
        out_specs=pl.BlockSpec((bn, D), lambda i: (i, 0)),
        out_shape=jax.ShapeDtypeStruct((N, D), jnp.float32),
    )(a2, d2, weights, h_bias.reshape(1, D))


def kernel(x, edge_attr, weights, h_bias, edge_index):
    a_flat, d_flat = _sc_aggregate(x, edge_attr, edge_index[0], edge_index[1])
    a2 = a_flat.reshape(NC, NPAD, D)[:, :N]
    d2 = d_flat.reshape(NC, NPAD, D)[:, :N]
    return _tc_finish(a2, d2, weights, h_bias)


# speedup vs baseline: 9.2975x; 1.3983x over previous
import functools

import jax
import jax.numpy as jnp
from jax import lax
from jax.experimental import pallas as pl
from jax.experimental.pallas import tpu as pltpu
from jax.experimental.pallas import tpu_sc as plsc

N = 10000
E = 320000
D = 128

NC = 2
NS = 16
CHUNK = 80
EPT = E // (NC * NS)
ITERS = EPT // CHUNK
PAIRS = (ITERS - 1) // 2
NPAD = 10240
ROWS_PT = NPAD // NS


def _sc_a_body(x_hbm, ea_hbm, srcp_hbm, dst3_hbm, zeros_hbm,
               outa_hbm,
               a_sh, dst_idx, src_a, src_b, xg_a, xg_b, ea_s,
               sem_ga, sem_gb, sem_sa, sem_sb):
    c = lax.axis_index("c")
    s = lax.axis_index("s")
    tid = c * NS + s
    base = tid * EPT
    row0 = s * ROWS_PT

    pltpu.sync_copy(zeros_hbm, xg_a)
    for k in range(ROWS_PT // CHUNK):
        pltpu.sync_copy(xg_a, a_sh.at[pl.ds(row0 + k * CHUNK, CHUNK)])
    pltpu.sync_copy(dst3_hbm.at[tid], dst_idx)
    plsc.subcore_barrier()

    def fire_src(j, sbuf, sem):
        pltpu.async_copy(srcp_hbm.at[tid, j], sbuf, sem)

    def fire_gather(sbuf, xbuf, sg):
        pltpu.async_copy(x_hbm.at[sbuf], xbuf, sg)

    def wait_ld(buf, sem):
        pltpu.make_async_copy(ea_hbm.at[pl.ds(0, CHUNK)], buf, sem).wait()

    def wait_src(sbuf, sem):
        pltpu.make_async_copy(srcp_hbm.at[0, 0], sbuf, sem).wait()

    def scatter(j, xbuf):
        pltpu.sync_copy(ea_hbm.at[pl.ds(base + j * CHUNK, CHUNK)], ea_s)
        pltpu.sync_copy(xbuf, a_sh.at[dst_idx.at[j]], add=True)
        pltpu.sync_copy(ea_s, a_sh.at[dst_idx.at[j]], add=True)

    fire_src(0, src_a, sem_sa)
    wait_src(src_a, sem_sa)
    fire_gather(src_a, xg_a, sem_ga)
    fire_src(1, src_b, sem_sb)

    def pair(i, carry):
        j0 = 2 * i
        wait_ld(xg_a, sem_ga)
        fire_src(j0 + 2, src_a, sem_sa)
        wait_src(src_b, sem_sb)
        fire_gather(src_b, xg_b, sem_gb)
        scatter(j0, xg_a)
        wait_ld(xg_b, sem_gb)
        fire_src(j0 + 3, src_b, sem_sb)
        wait_src(src_a, sem_sa)
        fire_gather(src_a, xg_a, sem_ga)
        scatter(j0 + 1, xg_b)
        return carry

    lax.fori_loop(0, PAIRS, pair, 0)
    wait_ld(xg_a, sem_ga)
    scatter(ITERS - 1, xg_a)
    wait_src(src_b, sem_sb)
    plsc.subcore_barrier()

    for k in range(ROWS_PT // CHUNK):
        r = row0 + k * CHUNK
        pltpu.sync_copy(a_sh.at[pl.ds(r, CHUNK)], xg_a)
        pltpu.sync_copy(xg_a, outa_hbm.at[pl.ds(c * NPAD + r, CHUNK)])


def _sc_d_body(dst3_hbm, zeros_hbm, ones_hbm,
               outd_hbm,
               d_sh, dst_idx, ones_v, sem):
    c = lax.axis_index("c")
    s = lax.axis_index("s")
    tid = c * NS + s
    row0 = s * ROWS_PT

    pltpu.sync_copy(zeros_hbm, ones_v)
    for k in range(ROWS_PT // CHUNK):
        pltpu.sync_copy(ones_v, d_sh.at[pl.ds(row0 + k * CHUNK, CHUNK)])
    pltpu.sync_copy(ones_hbm, ones_v)
    pltpu.sync_copy(dst3_hbm.at[tid], dst_idx)
    plsc.subcore_barrier()

    def step(j, carry):
        pltpu.sync_copy(ones_v, d_sh.at[dst_idx.at[j]], add=True)
        return carry

    lax.fori_loop(0, ITERS, step, 0)
    plsc.subcore_barrier()

    for k in range(ROWS_PT // CHUNK):
        r = row0 + k * CHUNK
        pltpu.sync_copy(d_sh.at[pl.ds(r, CHUNK)], ones_v)
        pltpu.sync_copy(ones_v, outd_hbm.at[pl.ds(c * NPAD + r, CHUNK)])


@jax.jit
def _sc_aggregate(x, edge_attr, srcp, dst3):
    zeros = jnp.zeros((CHUNK, D), jnp.float32)
    ones = jnp.ones((CHUNK, D), jnp.float32)
    mesh = plsc.VectorSubcoreMesh(core_axis_name="c", subcore_axis_name="s")
    a_fn = functools.partial(
        pl.kernel,
        _sc_a_body,
        out_type=[jax.ShapeDtypeStruct((NC * NPAD, D), jnp.float32)],
        mesh=mesh,
        scratch_types=[
            pltpu.VMEM_SHARED((NPAD, D), jnp.float32),
            pltpu.VMEM((ITERS, CHUNK), jnp.int32),
            pltpu.VMEM((CHUNK,), jnp.int32),
            pltpu.VMEM((CHUNK,), jnp.int32),
            pltpu.VMEM((CHUNK, D), jnp.float32),
            pltpu.VMEM((CHUNK, D), jnp.float32),
            pltpu.VMEM((CHUNK, D), jnp.float32),
            pltpu.SemaphoreType.DMA,
            pltpu.SemaphoreType.DMA,
            pltpu.SemaphoreType.DMA,
            pltpu.SemaphoreType.DMA,
        ],
    )()
    d_fn = functools.partial(
        pl.kernel,
        _sc_d_body,
        out_type=[jax.ShapeDtypeStruct((NC * NPAD, D), jnp.float32)],
        mesh=mesh,
        scratch_types=[
            pltpu.VMEM_SHARED((NPAD, D), jnp.float32),
            pltpu.VMEM((ITERS, CHUNK), jnp.int32),
            pltpu.VMEM((CHUNK, D), jnp.float32),
            pltpu.SemaphoreType.DMA,
        ],
    )()
    (a_flat,) = a_fn(x, edge_attr, srcp, dst3, zeros)
    (d_flat,) = d_fn(dst3, zeros, ones)
    return a_flat, d_flat


def _tc_body(a0_ref, a1_ref, d0_ref, d1_ref, w_ref, b_ref, out_ref):
    a = a0_ref[...] + a1_ref[...]
    deg = d0_ref[:, :1] + d1_ref[:, :1]
    norm = jnp.where(deg > 0.0, 1.0 / jnp.sqrt(deg), 0.0)
    out_ref[...] = (
        jnp.dot(a * norm, w_ref[...], preferred_element_type=jnp.float32)
        + b_ref[...]
    )


@jax.jit
def _tc_finish(a_flat, d_flat, weights, h_bias):
    bn = CHUNK
    off1 = NPAD // bn
    return pl.pallas_call(
        _tc_body,
        grid=(N // bn,),
        in_specs=[
            pl.BlockSpec((bn, D), lambda i: (i, 0)),
            pl.BlockSpec((bn, D), lambda i: (i + off1, 0)),
            pl.BlockSpec((bn, D), lambda i: (i, 0)),
            pl.BlockSpec((bn, D), lambda i: (i + off1, 0)),
            pl.BlockSpec((D, D), lambda i: (0, 0)),
            pl.BlockSpec((1, D), lambda i: (0, 0)),
        ],
        out_specs=pl.BlockSpec((bn, D), lambda i: (i, 0)),
        out_shape=jax.ShapeDtypeStruct((N, D), jnp.float32),
    )(a_flat, a_flat, d_flat, d_flat, weights, h_bias.reshape(1, D))


def kernel(x, edge_attr, weights, h_bias, edge_index):
    src3 = edge_index[0].reshape(NC * NS, ITERS, CHUNK)
    srcp = jnp.concatenate([src3, src3[:, :2]], axis=1)
    dst3 = edge_index[1].reshape(NC * NS, ITERS, CHUNK)
    a_flat, d_flat = _sc_aggregate(x, edge_attr, srcp, dst3)
    return _tc_finish(a_flat, d_flat, weights, h_bias)


# speedup vs baseline: 9.3099x; 1.0013x over previous
import functools

import jax
import jax.numpy as jnp
from jax import lax
from jax.experimental import pallas as pl
from jax.experimental.pallas import tpu as pltpu
from jax.experimental.pallas import tpu_sc as plsc

N = 10000
E = 320000
D = 128

NC = 2
NS = 16
CHUNK = 80
EPT = E // (NC * NS)
ITERS = EPT // CHUNK
PAIRS = (ITERS - 1) // 2
NBLK = N // CHUNK
BLK_ROUNDS = -(-NBLK // NS)


def _init_acc(sh_ref, zbuf, s):
    for k in range(BLK_ROUNDS):
        b = k * NS + s

        @pl.when(b < NBLK)
        def _():
            pltpu.sync_copy(zbuf, sh_ref.at[pl.ds(b * CHUNK, CHUNK)])


def _drain_acc(sh_ref, bbuf, out_hbm, c, s):
    for k in range(BLK_ROUNDS):
        b = k * NS + s

        @pl.when(b < NBLK)
        def _():
            pltpu.sync_copy(sh_ref.at[pl.ds(b * CHUNK, CHUNK)], bbuf)
            pltpu.sync_copy(bbuf, out_hbm.at[pl.ds(c * N + b * CHUNK, CHUNK)])


def _sc_a_body(x_hbm, ea_hbm, src3_hbm, dst3_hbm, zeros_hbm,
               outa_hbm,
               a_sh, idx_v, rows_v,
               sem_ga, sem_gb, sem_sa, sem_sb):
    c = lax.axis_index("c")
    s = lax.axis_index("s")
    tid = c * NS + s
    base = tid * EPT

    xg_a = rows_v.at[pl.ds(0 * CHUNK, CHUNK)]
    xg_b = rows_v.at[pl.ds(1 * CHUNK, CHUNK)]
    ea_s = rows_v.at[pl.ds(2 * CHUNK, CHUNK)]

    SRC_A, SRC_B = ITERS, ITERS + 1

    pltpu.sync_copy(zeros_hbm, xg_a)
    _init_acc(a_sh, xg_a, s)
    pltpu.sync_copy(dst3_hbm.at[tid], idx_v.at[pl.ds(0, ITERS)])
    plsc.subcore_barrier()

    def fire_src(j, slot, sem):
        pltpu.async_copy(src3_hbm.at[tid, pl.ds(j, 1)],
                         idx_v.at[pl.ds(slot, 1)], sem)

    def fire_gather(slot, xbuf, sg):
        pltpu.async_copy(x_hbm.at[idx_v.at[slot]], xbuf, sg)

    def wait_ld(buf, sem):
        pltpu.make_async_copy(ea_hbm.at[pl.ds(0, CHUNK)], buf, sem).wait()

    def wait_src(slot, sem):
        pltpu.make_async_copy(src3_hbm.at[0, pl.ds(0, 1)],
                              idx_v.at[pl.ds(slot, 1)], sem).wait()

    def scatter(j, xbuf):
        pltpu.sync_copy(ea_hbm.at[pl.ds(base + j * CHUNK, CHUNK)], ea_s)
        pltpu.sync_copy(xbuf, a_sh.at[idx_v.at[j]], add=True)
        pltpu.sync_copy(ea_s, a_sh.at[idx_v.at[j]], add=True)

    fire_src(0, SRC_A, sem_sa)
    wait_src(SRC_A, sem_sa)
    fire_gather(SRC_A, xg_a, sem_ga)
    fire_src(1, SRC_B, sem_sb)

    def pair(i, carry):
        j0 = 2 * i
        wait_ld(xg_a, sem_ga)
        fire_src(j0 + 2, SRC_A, sem_sa)
        wait_src(SRC_B, sem_sb)
        fire_gather(SRC_B, xg_b, sem_gb)
        scatter(j0, xg_a)
        wait_ld(xg_b, sem_gb)

        @pl.when(j0 + 3 < ITERS)
        def _():
            fire_src(j0 + 3, SRC_B, sem_sb)

        wait_src(SRC_A, sem_sa)
        fire_gather(SRC_A, xg_a, sem_ga)
        scatter(j0 + 1, xg_b)
        return carry

    lax.fori_loop(0, PAIRS, pair, 0)
    wait_ld(xg_a, sem_ga)
    scatter(ITERS - 1, xg_a)
    plsc.subcore_barrier()
    _drain_acc(a_sh, xg_a, outa_hbm, c, s)


def _sc_d_body(dst3_hbm, zeros_hbm, ones_hbm,
               outd_hbm,
               d_sh, dst_idx, ones_v, sem):
    c = lax.axis_index("c")
    s = lax.axis_index("s")
    tid = c * NS + s

    pltpu.sync_copy(zeros_hbm, ones_v)
    _init_acc(d_sh, ones_v, s)
    pltpu.sync_copy(ones_hbm, ones_v)
    pltpu.sync_copy(dst3_hbm.at[tid], dst_idx)
    plsc.subcore_barrier()

    def step(j, carry):
        pltpu.sync_copy(ones_v, d_sh.at[dst_idx.at[j]], add=True)
        return carry

    lax.fori_loop(0, ITERS, step, 0)
    plsc.subcore_barrier()
    _drain_acc(d_sh, ones_v, outd_hbm, c, s)


@jax.jit
def _sc_aggregate(x, edge_attr, src3, dst3):
    zeros = jnp.zeros((CHUNK, D), jnp.float32)
    ones = jnp.ones((CHUNK, D), jnp.float32)
    mesh = plsc.VectorSubcoreMesh(core_axis_name="c", subcore_axis_name="s")
    a_fn = functools.partial(
        pl.kernel,
        _sc_a_body,
        out_type=[jax.ShapeDtypeStruct((NC * N, D), jnp.float32)],
        mesh=mesh,
        scratch_types=[
            pltpu.VMEM_SHARED((N, D), jnp.float32),
            pltpu.VMEM((ITERS + 2, CHUNK), jnp.int32),
            pltpu.VMEM((3 * CHUNK, D), jnp.float32),
            pltpu.SemaphoreType.DMA,
            pltpu.SemaphoreType.DMA,
            pltpu.SemaphoreType.DMA,
            pltpu.SemaphoreType.DMA,
        ],
    )()
    d_fn = functools.partial(
        pl.kernel,
        _sc_d_body,
        out_type=[jax.ShapeDtypeStruct((NC * N, D), jnp.float32)],
        mesh=mesh,
        scratch_types=[
            pltpu.VMEM_SHARED((N, D), jnp.float32),
            pltpu.VMEM((ITERS, CHUNK), jnp.int32),
            pltpu.VMEM((CHUNK, D), jnp.float32),
            pltpu.SemaphoreType.DMA,
        ],
    )()
    (a_flat,) = a_fn(x, edge_attr, src3, dst3, zeros)
    (d_flat,) = d_fn(dst3, zeros, ones)
    return a_flat, d_flat


def _tc_body(a0_ref, a1_ref, d0_ref, d1_ref, w_ref, b_ref, out_ref):
    a = a0_ref[...] + a1_ref[...]
    deg = d0_ref[:, :1] + d1_ref[:, :1]
    norm = jnp.where(deg > 0.0, 1.0 / jnp.sqrt(deg), 0.0)
    out_ref[...] = (
        jnp.dot(a * norm, w_ref[...], preferred_element_type=jnp.float32)
        + b_ref[...]
    )


@jax.jit
def _tc_finish(a_flat, d_flat, weights, h_bias):
    bn = CHUNK
    off1 = N // bn
    return pl.pallas_call(
        _tc_body,
        grid=(N // bn,),
        in_specs=[
            pl.BlockSpec((bn, D), lambda i: (i, 0)),
            pl.BlockSpec((bn, D), lambda i: (i + off1, 0)),
            pl.BlockSpec((bn, D), lambda i: (i, 0)),
            pl.BlockSpec((bn, D), lambda i: (i + off1, 0)),
            pl.BlockSpec((D, D), lambda i: (0, 0)),
            pl.BlockSpec((1, D), lambda i: (0, 0)),
        ],
        out_specs=pl.BlockSpec((bn, D), lambda i: (i, 0)),
        out_shape=jax.ShapeDtypeStruct((N, D), jnp.float32),
    )(a_flat, a_flat, d_flat, d_flat, weights, h_bias.reshape(1, D))


def kernel(x, edge_attr, weights, h_bias, edge_index):
    src3 = edge_index[0].reshape(NC * NS, ITERS, CHUNK)
    dst3 = edge_index[1].reshape(NC * NS, ITERS, CHUNK)
    a_flat, d_flat = _sc_aggregate(x, edge_attr, src3, dst3)
    return _tc_finish(a_flat, d_flat, weights, h_bias)


# speedup vs baseline: 11.9662x; 1.2853x over previous
import functools

import jax
import jax.numpy as jnp
from jax import lax
from jax.experimental import pallas as pl
from jax.experimental.pallas import tpu as pltpu
from jax.experimental.pallas import tpu_sc as plsc

N = 10000
E = 320000
D = 128

NC = 2
NS = 16
CHUNK = 80
EPT = E // (NC * NS)
ITERS = EPT // CHUNK
PAIRS = (ITERS - 1) // 2
NBLK = N // CHUNK
BLK_ROUNDS = -(-NBLK // NS)


def _init_acc(sh_ref, zbuf, s):
    for k in range(BLK_ROUNDS):
        b = k * NS + s

        @pl.when(b < NBLK)
        def _():
            pltpu.sync_copy(zbuf, sh_ref.at[pl.ds(b * CHUNK, CHUNK)])


def _drain_acc(sh_ref, bbuf, out_hbm, c, s):
    for k in range(BLK_ROUNDS):
        b = k * NS + s

        @pl.when(b < NBLK)
        def _():
            pltpu.sync_copy(sh_ref.at[pl.ds(b * CHUNK, CHUNK)], bbuf)
            pltpu.sync_copy(bbuf, out_hbm.at[pl.ds(c * N + b * CHUNK, CHUNK)])


def _sc_body(x_hbm, ea_hbm, ei4_hbm, zeros_hbm, ones_hbm,
             outa_hbm, outd_hbm,
             a_sh, idx_v, rows_v,
             sem_g0, sem_g1, sem_g2, sem_s0, sem_s1, sem_s2):
    c = lax.axis_index("c")
    s = lax.axis_index("s")
    tid = c * NS + s
    base = tid * EPT

    slots = [rows_v.at[pl.ds(k * CHUNK, CHUNK)] for k in range(3)]
    ea_s = slots[0]

    pltpu.sync_copy(zeros_hbm, slots[0])
    _init_acc(a_sh, slots[0], s)
    pltpu.sync_copy(ei4_hbm.at[1, tid], idx_v.at[pl.ds(0, ITERS)])
    plsc.subcore_barrier()

    def fire_src(j, slot, sem):
        pltpu.async_copy(ei4_hbm.at[0, tid, pl.ds(j, 1)],
                         idx_v.at[pl.ds(slot, 1)], sem)

    def fire_gather(k, sg):
        pltpu.async_copy(x_hbm.at[idx_v.at[ITERS + k]], slots[k], sg)

    def wait_ld(buf, sem):
        pltpu.make_async_copy(ea_hbm.at[pl.ds(0, CHUNK)], buf, sem).wait()

    def wait_src(k, sem):
        pltpu.make_async_copy(ei4_hbm.at[0, 0, pl.ds(0, 1)],
                              idx_v.at[pl.ds(ITERS + k, 1)], sem).wait()

    def scatter(j, k):
        pltpu.sync_copy(slots[k], a_sh.at[idx_v.at[j]], add=True)
        pltpu.sync_copy(ea_hbm.at[pl.ds(base + j * CHUNK, CHUNK)], slots[k])
        pltpu.sync_copy(slots[k], a_sh.at[idx_v.at[j]], add=True)

    sem_g = [sem_g0, sem_g1, sem_g2]
    sem_s = [sem_s0, sem_s1, sem_s2]

    fire_src(0, ITERS + 0, sem_s[0])
    fire_src(1, ITERS + 1, sem_s[1])
    fire_src(2, ITERS + 2, sem_s[2])
    wait_src(0, sem_s[0])
    fire_gather(0, sem_g[0])
    wait_src(1, sem_s[1])
    fire_gather(1, sem_g[1])

    def sub(j, q):
        p = (q + 2) % 3
        wait_ld(slots[q], sem_g[q])

        @pl.when(j + 3 < ITERS)
        def _():
            fire_src(j + 3, ITERS + q, sem_s[q])

        @pl.when(j + 2 < ITERS)
        def _():
            wait_src(p, sem_s[p])
            fire_gather(p, sem_g[p])

        scatter(j, q)

    def triple(t, carry):
        j0 = 3 * t
        sub(j0, 0)
        sub(j0 + 1, 1)
        sub(j0 + 2, 2)
        return carry

    lax.fori_loop(0, ITERS // 3, triple, 0)
    sub(ITERS - 2, (ITERS - 2) % 3)
    sub(ITERS - 1, (ITERS - 1) % 3)
    plsc.subcore_barrier()
    _drain_acc(a_sh, slots[0], outa_hbm, c, s)

    pltpu.sync_copy(zeros_hbm, slots[1])
    _init_acc(a_sh, slots[1], s)
    pltpu.sync_copy(ones_hbm, ea_s)
    plsc.subcore_barrier()

    def dstep(j, carry):
        pltpu.sync_copy(ea_s, a_sh.at[idx_v.at[j]], add=True)
        return carry

    lax.fori_loop(0, ITERS, dstep, 0)
    plsc.subcore_barrier()
    _drain_acc(a_sh, slots[1], outd_hbm, c, s)


@jax.jit
def _sc_aggregate(x, edge_attr, ei4):
    zeros = jnp.zeros((CHUNK, D), jnp.float32)
    ones = jnp.ones((CHUNK, D), jnp.float32)
    mesh = plsc.VectorSubcoreMesh(core_axis_name="c", subcore_axis_name="s")
    fn = functools.partial(
        pl.kernel,
        _sc_body,
        out_type=[
            jax.ShapeDtypeStruct((NC * N, D), jnp.float32),
            jax.ShapeDtypeStruct((NC * N, D), jnp.float32),
        ],
        mesh=mesh,
        scratch_types=[
            pltpu.VMEM_SHARED((N, D), jnp.float32),
            pltpu.VMEM((ITERS + 3, CHUNK), jnp.int32),
            pltpu.VMEM((3 * CHUNK, D), jnp.float32),
            pltpu.SemaphoreType.DMA,
            pltpu.SemaphoreType.DMA,
            pltpu.SemaphoreType.DMA,
            pltpu.SemaphoreType.DMA,
            pltpu.SemaphoreType.DMA,
            pltpu.SemaphoreType.DMA,
        ],
    )()
    a_flat, d_flat = fn(x, edge_attr, ei4, zeros, ones)
    return a_flat, d_flat


def _tc_body(a0_ref, a1_ref, d0_ref, d1_ref, w_ref, b_ref, out_ref):
    a = a0_ref[...] + a1_ref[...]
    deg = d0_ref[:, :1] + d1_ref[:, :1]
    norm = jnp.where(deg > 0.0, 1.0 / jnp.sqrt(deg), 0.0)
    out_ref[...] = (
        jnp.dot(a * norm, w_ref[...], preferred_element_type=jnp.float32)
        + b_ref[...]
    )


@jax.jit
def _tc_finish(a_flat, d_flat, weights, h_bias):
    bn = 400
    off1 = N // bn
    return pl.pallas_call(
        _tc_body,
        grid=(N // bn,),
        in_specs=[
            pl.BlockSpec((bn, D), lambda i: (i, 0)),
            pl.BlockSpec((bn, D), lambda i: (i + off1, 0)),
            pl.BlockSpec((bn, D), lambda i: (i, 0)),
            pl.BlockSpec((bn, D), lambda i: (i + off1, 0)),
            pl.BlockSpec((D, D), lambda i: (0, 0)),
            pl.BlockSpec((1, D), lambda i: (0, 0)),
        ],
        out_specs=pl.BlockSpec((bn, D), lambda i: (i, 0)),
        out_shape=jax.ShapeDtypeStruct((N, D), jnp.float32),
    )(a_flat, a_flat, d_flat, d_flat, weights, h_bias.reshape(1, D))


def kernel(x, edge_attr, weights, h_bias, edge_index):
    ei4 = edge_index.reshape(2, NC * NS, ITERS, CHUNK)
    a_flat, d_flat = _sc_aggregate(x, edge_attr, ei4)
    return _tc_finish(a_flat, d_flat, weights, h_bias)
